# baseline (speedup 1.0000x reference)
import functools

import jax
import jax.numpy as jnp
import numpy as np
from jax import lax
from jax.experimental import pallas as pl
from jax.experimental.pallas import tpu as pltpu
from jax.experimental.pallas import tpu_sc as plsc

_BS = 2
_Q = 10000
_D = 256
_NH = 8
_NP = 4
_HD = 32
_SIDE = 100
_ROWS = _BS * _Q
_T = 400
_GRID = _ROWS // _T
_NW = 32
_CH = 8
_NCHUNK = _ROWS // _CH


def _mm_bias_body(x_ref, w_ref, b_ref, o_ref):
    o_ref[...] = (jnp.dot(x_ref[...], w_ref[...],
                          preferred_element_type=jnp.float32)
                  + b_ref[...]).astype(jnp.bfloat16)


def _mm_bias(x, w, b):
    return pl.pallas_call(
        _mm_bias_body,
        grid=(_GRID,),
        in_specs=[
            pl.BlockSpec((_T, _D), lambda i: (i, 0)),
            pl.BlockSpec((_D, _D), lambda i: (0, 0)),
            pl.BlockSpec((1, _D), lambda i: (0, 0)),
        ],
        out_specs=pl.BlockSpec((_T, _D), lambda i: (i, 0)),
        out_shape=jax.ShapeDtypeStruct((_ROWS, _D), jnp.bfloat16),
    )(x, w, b)


def _mm_bias_res_body(x_ref, w_ref, b_ref, id_ref, o_ref):
    o_ref[...] = (jnp.dot(x_ref[...], w_ref[...],
                          preferred_element_type=jnp.float32)
                  + b_ref[...] + id_ref[...])


def _mm_bias_res(x, w, b, ident):
    return pl.pallas_call(
        _mm_bias_res_body,
        grid=(_GRID,),
        in_specs=[
            pl.BlockSpec((_T, _D), lambda i: (i, 0)),
            pl.BlockSpec((_D, _D), lambda i: (0, 0)),
            pl.BlockSpec((1, _D), lambda i: (0, 0)),
            pl.BlockSpec((_T, _D), lambda i: (i, 0)),
        ],
        out_specs=pl.BlockSpec((_T, _D), lambda i: (i, 0)),
        out_shape=jax.ShapeDtypeStruct((_ROWS, _D), jnp.float32),
    )(x, w, b, ident)


def _prep_body(q_ref, r_ref, wcat_ref, bcat_ref, s_ref, idx_ref, cw_ref):
    t = pl.program_id(0)
    b = t // (_GRID // _BS)
    s = (jnp.dot(q_ref[...], wcat_ref[...],
                 preferred_element_type=jnp.float32) + bcat_ref[...])
    ox = s[:, 0:32]
    oy = s[:, 32:64]
    lg = s[:, 64:96]
    e = jnp.exp(lg)
    denom = jnp.dot(e, s_ref[...], preferred_element_type=jnp.float32)
    attn = e / denom
    rx = r_ref[:, 0:1]
    ry = r_ref[:, 1:2]
    gx = rx * float(_SIDE) - 0.5 + ox
    gy = ry * float(_SIDE) - 0.5 + oy
    x0 = jnp.floor(gx)
    fx = gx - x0
    y0 = jnp.floor(gy)
    fy = gy - y0
    lane = lax.broadcasted_iota(jnp.int32, (_T, 128), 1)
    h = (lane % 32) // _NP
    base = b * (_Q * _NH) + h
    lim = float(_SIDE - 1)

    xf = jnp.concatenate([x0, x0 + 1.0, x0, x0 + 1.0], axis=1)
    yf = jnp.concatenate([y0, y0, y0 + 1.0, y0 + 1.0], axis=1)
    wxy = jnp.concatenate(
        [(1.0 - fx) * (1.0 - fy), fx * (1.0 - fy),
         (1.0 - fx) * fy, fx * fy], axis=1)
    attn4 = jnp.concatenate([attn, attn, attn, attn], axis=1)
    v = (xf >= 0.0) & (xf <= lim) & (yf >= 0.0) & (yf <= lim)
    xi = jnp.clip(xf, 0.0, lim).astype(jnp.int32)
    yi = jnp.clip(yf, 0.0, lim).astype(jnp.int32)
    idx_ref[...] = base + (yi * _SIDE + xi) * _NH
    cw_ref[...] = jnp.where(v, wxy * attn4, 0.0)


def _prep(q, r, wcat, bcat, smat):
    return pl.pallas_call(
        _prep_body,
        grid=(_GRID,),
        in_specs=[
            pl.BlockSpec((_T, _D), lambda i: (i, 0)),
            pl.BlockSpec((_T, 2), lambda i: (i, 0)),
            pl.BlockSpec((_D, 96), lambda i: (0, 0)),
            pl.BlockSpec((1, 96), lambda i: (0, 0)),
            pl.BlockSpec((32, 32), lambda i: (0, 0)),
        ],
        out_specs=[
            pl.BlockSpec((_T, 128), lambda i: (i, 0)),
            pl.BlockSpec((_T, 128), lambda i: (i, 0)),
        ],
        out_shape=[
            jax.ShapeDtypeStruct((_ROWS, 128), jnp.int32),
            jax.ShapeDtypeStruct((_ROWS, 128), jnp.float32),
        ],
    )(q, r, wcat, bcat, smat)


def _bcast_lane(v, j):
    idx = jnp.full((16, 1), j, dtype=jnp.int32)
    dn = lax.GatherDimensionNumbers(
        offset_dims=(), collapsed_slice_dims=(0,), start_index_map=(0,))
    return lax.gather(v, idx, dn, (1,),
                      mode=lax.GatherScatterMode.PROMISE_IN_BOUNDS)


def _sc_body(table_hbm, idx_hbm, cw_hbm, out_hbm,
             idx_v, cw_v, rows_v, out_v, gsem, isem, osem):
    wid = lax.axis_index("s") * 2 + lax.axis_index("c")
    nk = (_NCHUNK - wid + _NW - 1) // _NW

    def row0_of(k):
        return (wid + k * _NW) * _CH

    def copy_in_descs(k):
        buf = k % 2
        r0 = row0_of(k)
        return (
            pltpu.make_async_copy(idx_hbm.at[pl.ds(r0, _CH)],
                                  idx_v.at[buf], isem),
            pltpu.make_async_copy(cw_hbm.at[pl.ds(r0, _CH)],
                                  cw_v.at[buf], isem),
        )

    def gather_descs(k):
        buf = k % 2
        return [pltpu.make_async_copy(table_hbm.at[idx_v.at[buf, j]],
                                      rows_v.at[buf, j], gsem)
                for j in range(_CH)]

    def out_desc(k):
        buf = k % 2
        return pltpu.make_async_copy(
            out_v.at[buf], out_hbm.at[pl.ds(row0_of(k) * _NH, _CH * _NH)],
            osem)

    def start_all(descs):
        for d in descs:
            d.start()

    def wait_all(descs):
        for d in descs:
            d.wait()

    start_all(copy_in_descs(0))
    wait_all(copy_in_descs(0))
    start_all(gather_descs(0))
    start_all(copy_in_descs(1))

    def step(k, carry):
        buf = k % 2

        @pl.when(k + 1 < nk)
        def _():
            wait_all(copy_in_descs(k + 1))
            start_all(gather_descs(k + 1))

        wait_all(gather_descs(k))

        @pl.when(k >= 2)
        def _():
            out_desc(k - 2).wait()

        def qbody(q, carry2):
            wv = [cw_v[buf, q, pl.ds(kk * 16, 16)] for kk in range(8)]
            for h in range(_NH):
                acc0 = jnp.zeros((16,), jnp.float32)
                acc1 = jnp.zeros((16,), jnp.float32)
                for c4 in range(4):
                    for p in range(_NP):
                        ln = c4 * 32 + h * _NP + p
                        w = _bcast_lane(wv[ln // 16], ln % 16)
                        r32 = rows_v[buf, q, ln, :]
                        ev = lax.bitcast_convert_type(
                            lax.shift_left(r32, 16), jnp.float32)
                        od = lax.bitcast_convert_type(
                            lax.bitwise_and(r32, jnp.int32(-65536)),
                            jnp.float32)
                        acc0 = acc0 + w * ev
                        acc1 = acc1 + w * od
                out_v[buf, q * _NH + h, pl.ds(0, 16)] = acc0
                out_v[buf, q * _NH + h, pl.ds(16, 16)] = acc1
            return carry2

        lax.fori_loop(0, _CH, qbody, 0)
        out_desc(k).start()

        @pl.when(k + 2 < nk)
        def _():
            start_all(copy_in_descs(k + 2))

        return carry

    lax.fori_loop(0, nk, step, 0)
    out_desc(nk - 2).wait()
    out_desc(nk - 1).wait()


@functools.cache
def _sc_gather_fn():
    return functools.partial(
        pl.kernel,
        mesh=plsc.VectorSubcoreMesh(core_axis_name="c", subcore_axis_name="s"),
        compiler_params=pltpu.CompilerParams(use_tc_tiling_on_sc=False),
        out_type=jax.ShapeDtypeStruct((_ROWS * _NH, _HD), jnp.float32),
        scratch_types=[
            pltpu.VMEM((2, _CH, 128), jnp.int32),
            pltpu.VMEM((2, _CH, 128), jnp.float32),
            pltpu.VMEM((2, _CH, 128, _HD // 2), jnp.int32),
            pltpu.VMEM((2, _CH * _NH, _HD), jnp.float32),
            pltpu.SemaphoreType.DMA,
            pltpu.SemaphoreType.DMA,
            pltpu.SemaphoreType.DMA,
        ],
    )(_sc_body)


def kernel(query, value, ref_2d, spatial_shapes, level_start_index,
           W_off, b_off, W_attn, b_attn, W_val, b_val, W_out, b_out):
    q2 = query.reshape(_ROWS, _D)
    v2 = value.reshape(_ROWS, _D)
    r2 = ref_2d.reshape(_ROWS, 2)

    wo = W_off.reshape(_NH, _NP, 2, _D)
    wcat = jnp.concatenate(
        [wo[:, :, 0, :].reshape(32, _D),
         wo[:, :, 1, :].reshape(32, _D),
         W_attn], axis=0).T
    bo = b_off.reshape(_NH, _NP, 2)
    bcat = jnp.concatenate(
        [bo[:, :, 0].reshape(32), bo[:, :, 1].reshape(32), b_attn],
        axis=0).reshape(1, 96)
    smat = jnp.asarray(
        np.kron(np.eye(_NH, dtype=np.float32),
                np.ones((_NP, _NP), dtype=np.float32)))

    table = _mm_bias(v2, W_val.T, b_val.reshape(1, _D))
    tbl_i32 = lax.bitcast_convert_type(
        table.reshape(_ROWS * _NH, _HD // 2, 2), jnp.int32)
    idx, cw = _prep(q2, r2, wcat, bcat, smat)
    sampled = _sc_gather_fn()(tbl_i32, idx, cw)
    perm = np.concatenate(
        [h * _HD + np.concatenate([np.arange(0, _HD, 2), np.arange(1, _HD, 2)])
         for h in range(_NH)])
    out = _mm_bias_res(sampled.reshape(_ROWS, _D), W_out.T[perm],
                       b_out.reshape(1, _D), q2)
    return out.reshape(_BS, _Q, _D)


# speedup vs baseline: 11.1333x; 11.1333x over previous
import functools

import jax
import jax.numpy as jnp
import numpy as np
from jax import lax
from jax.experimental import pallas as pl
from jax.experimental.pallas import tpu as pltpu
from jax.experimental.pallas import tpu_sc as plsc

_BS = 2
_Q = 10000
_D = 256
_NH = 8
_NP = 4
_HD = 32
_SIDE = 100
_ROWS = _BS * _Q
_T = 400
_GRID = _ROWS // _T
_NW = 32
_CH = 8
_NCHUNK = _ROWS // _CH


def _round_bf16_bits(x):
    xi = lax.bitcast_convert_type(x, jnp.int32)
    r = lax.shift_right_arithmetic(
        xi + 0x7FFF + lax.bitwise_and(lax.shift_right_arithmetic(xi, 16), 1),
        16)
    return lax.bitwise_and(r, 0xFFFF)


def _mm_bias_body(x_ref, w_ref, b_ref, o_ref):
    m = (jnp.dot(x_ref[...], w_ref[...],
                 preferred_element_type=jnp.float32) + b_ref[...])
    parts = []
    for h in range(_NH):
        lo = _round_bf16_bits(m[:, h * _HD:h * _HD + 16])
        hi = _round_bf16_bits(m[:, h * _HD + 16:(h + 1) * _HD])
        parts.append(lax.bitwise_or(lo, lax.shift_left(hi, 16)))
    o_ref[...] = jnp.concatenate(parts, axis=1)


def _mm_bias(x, w, b):
    return pl.pallas_call(
        _mm_bias_body,
        grid=(_GRID,),
        in_specs=[
            pl.BlockSpec((_T, _D), lambda i: (i, 0)),
            pl.BlockSpec((_D, _D), lambda i: (0, 0)),
            pl.BlockSpec((1, _D), lambda i: (0, 0)),
        ],
        out_specs=pl.BlockSpec((_T, _D // 2), lambda i: (i, 0)),
        out_shape=jax.ShapeDtypeStruct((_ROWS, _D // 2), jnp.int32),
    )(x, w, b)


def _mm_bias_res_body(x_ref, w_ref, b_ref, id_ref, o_ref):
    o_ref[...] = (jnp.dot(x_ref[...], w_ref[...],
                          preferred_element_type=jnp.float32)
                  + b_ref[...] + id_ref[...])


def _mm_bias_res(x, w, b, ident):
    return pl.pallas_call(
        _mm_bias_res_body,
        grid=(_GRID,),
        in_specs=[
            pl.BlockSpec((_T, _D), lambda i: (i, 0)),
            pl.BlockSpec((_D, _D), lambda i: (0, 0)),
            pl.BlockSpec((1, _D), lambda i: (0, 0)),
            pl.BlockSpec((_T, _D), lambda i: (i, 0)),
        ],
        out_specs=pl.BlockSpec((_T, _D), lambda i: (i, 0)),
        out_shape=jax.ShapeDtypeStruct((_ROWS, _D), jnp.float32),
    )(x, w, b, ident)


def _prep_body(q_ref, r_ref, wcat_ref, bcat_ref, s_ref, idx_ref, cw_ref):
    t = pl.program_id(0)
    b = t // (_GRID // _BS)
    s = (jnp.dot(q_ref[...], wcat_ref[...],
                 preferred_element_type=jnp.float32) + bcat_ref[...])
    ox = s[:, 0:32]
    oy = s[:, 32:64]
    lg = s[:, 64:96]
    e = jnp.exp(lg)
    denom = jnp.dot(e, s_ref[...], preferred_element_type=jnp.float32)
    attn = e / denom
    rx = r_ref[:, 0:1]
    ry = r_ref[:, 1:2]
    gx = rx * float(_SIDE) - 0.5 + ox
    gy = ry * float(_SIDE) - 0.5 + oy
    x0 = jnp.floor(gx)
    fx = gx - x0
    y0 = jnp.floor(gy)
    fy = gy - y0
    lane = lax.broadcasted_iota(jnp.int32, (_T, 128), 1)
    h = (lane % 32) // _NP
    base = b * (_Q * _NH) + h
    lim = float(_SIDE - 1)

    xf = jnp.concatenate([x0, x0 + 1.0, x0, x0 + 1.0], axis=1)
    yf = jnp.concatenate([y0, y0, y0 + 1.0, y0 + 1.0], axis=1)
    wxy = jnp.concatenate(
        [(1.0 - fx) * (1.0 - fy), fx * (1.0 - fy),
         (1.0 - fx) * fy, fx * fy], axis=1)
    attn4 = jnp.concatenate([attn, attn, attn, attn], axis=1)
    v = (xf >= 0.0) & (xf <= lim) & (yf >= 0.0) & (yf <= lim)
    xi = jnp.clip(xf, 0.0, lim).astype(jnp.int32)
    yi = jnp.clip(yf, 0.0, lim).astype(jnp.int32)
    idx_ref[...] = base + (yi * _SIDE + xi) * _NH
    cw_ref[...] = jnp.where(v, wxy * attn4, 0.0)


def _prep(q, r, wcat, bcat, smat):
    return pl.pallas_call(
        _prep_body,
        grid=(_GRID,),
        in_specs=[
            pl.BlockSpec((_T, _D), lambda i: (i, 0)),
            pl.BlockSpec((_T, 2), lambda i: (i, 0)),
            pl.BlockSpec((_D, 96), lambda i: (0, 0)),
            pl.BlockSpec((1, 96), lambda i: (0, 0)),
            pl.BlockSpec((32, 32), lambda i: (0, 0)),
        ],
        out_specs=[
            pl.BlockSpec((_T, 128), lambda i: (i, 0)),
            pl.BlockSpec((_T, 128), lambda i: (i, 0)),
        ],
        out_shape=[
            jax.ShapeDtypeStruct((_ROWS, 128), jnp.int32),
            jax.ShapeDtypeStruct((_ROWS, 128), jnp.float32),
        ],
    )(q, r, wcat, bcat, smat)


def _bcast_lane(v, j):
    idx = jnp.full((16, 1), j, dtype=jnp.int32)
    dn = lax.GatherDimensionNumbers(
        offset_dims=(), collapsed_slice_dims=(0,), start_index_map=(0,))
    return lax.gather(v, idx, dn, (1,),
                      mode=lax.GatherScatterMode.PROMISE_IN_BOUNDS)


def _sc_body(table_hbm, idx_hbm, cw_hbm, out_hbm,
             idx_v, cw_v, rows_v, out_v, gsem, isem, osem):
    wid = lax.axis_index("s") * 2 + lax.axis_index("c")
    nk = (_NCHUNK - wid + _NW - 1) // _NW

    def row0_of(k):
        return (wid + k * _NW) * _CH

    def copy_in_descs(k):
        buf = k % 2
        r0 = row0_of(k)
        return (
            pltpu.make_async_copy(idx_hbm.at[pl.ds(r0, _CH)],
                                  idx_v.at[buf], isem),
            pltpu.make_async_copy(cw_hbm.at[pl.ds(r0, _CH)],
                                  cw_v.at[buf], isem),
        )

    def gather_descs(k):
        buf = k % 2
        return [pltpu.make_async_copy(table_hbm.at[idx_v.at[buf, j]],
                                      rows_v.at[buf, j], gsem)
                for j in range(_CH)]

    def out_desc(k):
        buf = k % 2
        return pltpu.make_async_copy(
            out_v.at[buf], out_hbm.at[pl.ds(row0_of(k) * _NH, _CH * _NH)],
            osem)

    def start_all(descs):
        for d in descs:
            d.start()

    def wait_all(descs):
        for d in descs:
            d.wait()

    start_all(copy_in_descs(0))
    wait_all(copy_in_descs(0))
    start_all(gather_descs(0))
    start_all(copy_in_descs(1))

    def step(k, carry):
        buf = k % 2

        @pl.when(k + 1 < nk)
        def _():
            wait_all(copy_in_descs(k + 1))
            start_all(gather_descs(k + 1))

        wait_all(gather_descs(k))

        @pl.when(k >= 2)
        def _():
            out_desc(k - 2).wait()

        def qbody(q, carry2):
            wv = [cw_v[buf, q, pl.ds(kk * 16, 16)] for kk in range(8)]
            for h in range(_NH):
                acc0 = jnp.zeros((16,), jnp.float32)
                acc1 = jnp.zeros((16,), jnp.float32)
                for c4 in range(4):
                    for p in range(_NP):
                        ln = c4 * 32 + h * _NP + p
                        w = _bcast_lane(wv[ln // 16], ln % 16)
                        r32 = rows_v[buf, q, ln, :]
                        ev = lax.bitcast_convert_type(
                            lax.shift_left(r32, 16), jnp.float32)
                        od = lax.bitcast_convert_type(
                            lax.bitwise_and(r32, jnp.int32(-65536)),
                            jnp.float32)
                        acc0 = acc0 + w * ev
                        acc1 = acc1 + w * od
                out_v[buf, q * _NH + h, pl.ds(0, 16)] = acc0
                out_v[buf, q * _NH + h, pl.ds(16, 16)] = acc1
            return carry2

        lax.fori_loop(0, _CH, qbody, 0)
        out_desc(k).start()

        @pl.when(k + 2 < nk)
        def _():
            start_all(copy_in_descs(k + 2))

        return carry

    lax.fori_loop(0, nk, step, 0)
    out_desc(nk - 2).wait()
    out_desc(nk - 1).wait()


@functools.cache
def _sc_gather_fn():
    return functools.partial(
        pl.kernel,
        mesh=plsc.VectorSubcoreMesh(core_axis_name="c", subcore_axis_name="s"),
        compiler_params=pltpu.CompilerParams(use_tc_tiling_on_sc=False),
        out_type=jax.ShapeDtypeStruct((_ROWS * _NH, _HD), jnp.float32),
        scratch_types=[
            pltpu.VMEM((2, _CH, 128), jnp.int32),
            pltpu.VMEM((2, _CH, 128), jnp.float32),
            pltpu.VMEM((2, _CH, 128, _HD // 2), jnp.int32),
            pltpu.VMEM((2, _CH * _NH, _HD), jnp.float32),
            pltpu.SemaphoreType.DMA,
            pltpu.SemaphoreType.DMA,
            pltpu.SemaphoreType.DMA,
        ],
    )(_sc_body)


def kernel(query, value, ref_2d, spatial_shapes, level_start_index,
           W_off, b_off, W_attn, b_attn, W_val, b_val, W_out, b_out):
    q2 = query.reshape(_ROWS, _D)
    v2 = value.reshape(_ROWS, _D)
    r2 = ref_2d.reshape(_ROWS, 2)

    wo = W_off.reshape(_NH, _NP, 2, _D)
    wcat = jnp.concatenate(
        [wo[:, :, 0, :].reshape(32, _D),
         wo[:, :, 1, :].reshape(32, _D),
         W_attn], axis=0).T
    bo = b_off.reshape(_NH, _NP, 2)
    bcat = jnp.concatenate(
        [bo[:, :, 0].reshape(32), bo[:, :, 1].reshape(32), b_attn],
        axis=0).reshape(1, 96)
    smat = jnp.asarray(
        np.kron(np.eye(_NH, dtype=np.float32),
                np.ones((_NP, _NP), dtype=np.float32)))

    table = _mm_bias(v2, W_val.T, b_val.reshape(1, _D))
    idx, cw = _prep(q2, r2, wcat, bcat, smat)
    sampled = _sc_gather_fn()(table.reshape(_ROWS * _NH, _HD // 2), idx, cw)
    out = _mm_bias_res(sampled.reshape(_ROWS, _D), W_out.T,
                       b_out.reshape(1, _D), q2)
    return out.reshape(_BS, _Q, _D)


# speedup vs baseline: 11.4511x; 1.0285x over previous
import functools

import jax
import jax.numpy as jnp
import numpy as np
from jax import lax
from jax.experimental import pallas as pl
from jax.experimental.pallas import tpu as pltpu
from jax.experimental.pallas import tpu_sc as plsc

_BS = 2
_Q = 10000
_D = 256
_NH = 8
_NP = 4
_HD = 32
_SIDE = 100
_ROWS = _BS * _Q
_T = 400
_GRID = _ROWS // _T
_NW = 32
_CH = 8
_NCHUNK = _ROWS // _CH


def _mm_bias_body(x_ref, w_ref, b_ref, o_ref):
    o_ref[...] = (jnp.dot(x_ref[...], w_ref[...],
                          preferred_element_type=jnp.float32) + b_ref[...])


def _mm_bias(x, w, b):
    return pl.pallas_call(
        _mm_bias_body,
        grid=(_GRID,),
        in_specs=[
            pl.BlockSpec((_T, _D), lambda i: (i, 0)),
            pl.BlockSpec((_D, _D), lambda i: (0, 0)),
            pl.BlockSpec((1, _D), lambda i: (0, 0)),
        ],
        out_specs=pl.BlockSpec((_T, _D), lambda i: (i, 0)),
        out_shape=jax.ShapeDtypeStruct((_ROWS, _D), jnp.float32),
    )(x, w, b)


def _mm_bias_res_body(x_ref, w_ref, b_ref, id_ref, o_ref):
    o_ref[...] = (jnp.dot(x_ref[...], w_ref[...],
                          preferred_element_type=jnp.float32)
                  + b_ref[...] + id_ref[...])


def _mm_bias_res(x, w, b, ident):
    return pl.pallas_call(
        _mm_bias_res_body,
        grid=(_GRID,),
        in_specs=[
            pl.BlockSpec((_T, _D), lambda i: (i, 0)),
            pl.BlockSpec((_D, _D), lambda i: (0, 0)),
            pl.BlockSpec((1, _D), lambda i: (0, 0)),
            pl.BlockSpec((_T, _D), lambda i: (i, 0)),
        ],
        out_specs=pl.BlockSpec((_T, _D), lambda i: (i, 0)),
        out_shape=jax.ShapeDtypeStruct((_ROWS, _D), jnp.float32),
    )(x, w, b, ident)


def _prep_body(q_ref, r_ref, wcat_ref, bcat_ref, s_ref, idx_ref, cw_ref):
    t = pl.program_id(0)
    b = t // (_GRID // _BS)
    s = (jnp.dot(q_ref[...], wcat_ref[...],
                 preferred_element_type=jnp.float32) + bcat_ref[...])
    ox = s[:, 0:32]
    oy = s[:, 32:64]
    lg = s[:, 64:96]
    e = jnp.exp(lg)
    denom = jnp.dot(e, s_ref[...], preferred_element_type=jnp.float32)
    attn = e / denom
    rx = r_ref[:, 0:1]
    ry = r_ref[:, 1:2]
    gx = rx * float(_SIDE) - 0.5 + ox
    gy = ry * float(_SIDE) - 0.5 + oy
    x0 = jnp.floor(gx)
    fx = gx - x0
    y0 = jnp.floor(gy)
    fy = gy - y0
    lane = lax.broadcasted_iota(jnp.int32, (_T, 128), 1)
    h = (lane % 32) // _NP
    base = b * (_Q * _NH) + h
    lim = float(_SIDE - 1)

    xf = jnp.concatenate([x0, x0 + 1.0, x0, x0 + 1.0], axis=1)
    yf = jnp.concatenate([y0, y0, y0 + 1.0, y0 + 1.0], axis=1)
    wxy = jnp.concatenate(
        [(1.0 - fx) * (1.0 - fy), fx * (1.0 - fy),
         (1.0 - fx) * fy, fx * fy], axis=1)
    attn4 = jnp.concatenate([attn, attn, attn, attn], axis=1)
    v = (xf >= 0.0) & (xf <= lim) & (yf >= 0.0) & (yf <= lim)
    xi = jnp.clip(xf, 0.0, lim).astype(jnp.int32)
    yi = jnp.clip(yf, 0.0, lim).astype(jnp.int32)
    idx_ref[...] = base + (yi * _SIDE + xi) * _NH
    cw_ref[...] = jnp.where(v, wxy * attn4, 0.0)


def _prep(q, r, wcat, bcat, smat):
    return pl.pallas_call(
        _prep_body,
        grid=(_GRID,),
        in_specs=[
            pl.BlockSpec((_T, _D), lambda i: (i, 0)),
            pl.BlockSpec((_T, 2), lambda i: (i, 0)),
            pl.BlockSpec((_D, 96), lambda i: (0, 0)),
            pl.BlockSpec((1, 96), lambda i: (0, 0)),
            pl.BlockSpec((32, 32), lambda i: (0, 0)),
        ],
        out_specs=[
            pl.BlockSpec((_T, 128), lambda i: (i, 0)),
            pl.BlockSpec((_T, 128), lambda i: (i, 0)),
        ],
        out_shape=[
            jax.ShapeDtypeStruct((_ROWS, 128), jnp.int32),
            jax.ShapeDtypeStruct((_ROWS, 128), jnp.float32),
        ],
    )(q, r, wcat, bcat, smat)


def _bcast_lane(v, j):
    idx = jnp.full((16, 1), j, dtype=jnp.int32)
    dn = lax.GatherDimensionNumbers(
        offset_dims=(), collapsed_slice_dims=(0,), start_index_map=(0,))
    return lax.gather(v, idx, dn, (1,),
                      mode=lax.GatherScatterMode.PROMISE_IN_BOUNDS)


def _sc_body(table_hbm, idx_hbm, cw_hbm, out_hbm,
             idx_v, cw_v, rows_v, out_v, gsem, isem, osem):
    wid = lax.axis_index("s") * 2 + lax.axis_index("c")
    nk = (_NCHUNK - wid + _NW - 1) // _NW

    def row0_of(k):
        return (wid + k * _NW) * _CH

    def copy_in_descs(k):
        buf = k % 2
        r0 = row0_of(k)
        return (
            pltpu.make_async_copy(idx_hbm.at[pl.ds(r0, _CH)],
                                  idx_v.at[buf], isem),
            pltpu.make_async_copy(cw_hbm.at[pl.ds(r0, _CH)],
                                  cw_v.at[buf], isem),
        )

    def gather_descs(k):
        buf = k % 2
        return [pltpu.make_async_copy(table_hbm.at[idx_v.at[buf, j]],
                                      rows_v.at[buf, j], gsem)
                for j in range(_CH)]

    def out_desc(k):
        buf = k % 2
        return pltpu.make_async_copy(
            out_v.at[buf], out_hbm.at[pl.ds(row0_of(k) * _NH, _CH * _NH)],
            osem)

    def start_all(descs):
        for d in descs:
            d.start()

    def wait_all(descs):
        for d in descs:
            d.wait()

    start_all(copy_in_descs(0))
    wait_all(copy_in_descs(0))
    start_all(gather_descs(0))
    start_all(copy_in_descs(1))

    def step(k, carry):
        buf = k % 2

        @pl.when(k + 1 < nk)
        def _():
            wait_all(copy_in_descs(k + 1))
            start_all(gather_descs(k + 1))

        wait_all(gather_descs(k))

        @pl.when(k >= 2)
        def _():
            out_desc(k - 2).wait()

        def qbody(q, carry2):
            wv = [cw_v[buf, q, pl.ds(kk * 16, 16)] for kk in range(8)]
            for h in range(_NH):
                acc0 = jnp.zeros((16,), jnp.float32)
                acc1 = jnp.zeros((16,), jnp.float32)
                for c4 in range(4):
                    for p in range(_NP):
                        ln = c4 * 32 + h * _NP + p
                        w = _bcast_lane(wv[ln // 16], ln % 16)
                        acc0 = acc0 + w * rows_v[buf, q, ln, pl.ds(0, 16)]
                        acc1 = acc1 + w * rows_v[buf, q, ln, pl.ds(16, 16)]
                out_v[buf, q * _NH + h, pl.ds(0, 16)] = acc0
                out_v[buf, q * _NH + h, pl.ds(16, 16)] = acc1
            return carry2

        lax.fori_loop(0, _CH, qbody, 0)
        out_desc(k).start()

        @pl.when(k + 2 < nk)
        def _():
            start_all(copy_in_descs(k + 2))

        return carry

    lax.fori_loop(0, nk, step, 0)
    out_desc(nk - 2).wait()
    out_desc(nk - 1).wait()


@functools.cache
def _sc_gather_fn():
    return functools.partial(
        pl.kernel,
        mesh=plsc.VectorSubcoreMesh(core_axis_name="c", subcore_axis_name="s"),
        compiler_params=pltpu.CompilerParams(use_tc_tiling_on_sc=False),
        out_type=jax.ShapeDtypeStruct((_ROWS * _NH, _HD), jnp.float32),
        scratch_types=[
            pltpu.VMEM((2, _CH, 128), jnp.int32),
            pltpu.VMEM((2, _CH, 128), jnp.float32),
            pltpu.VMEM((2, _CH, 128, _HD), jnp.float32),
            pltpu.VMEM((2, _CH * _NH, _HD), jnp.float32),
            pltpu.SemaphoreType.DMA,
            pltpu.SemaphoreType.DMA,
            pltpu.SemaphoreType.DMA,
        ],
    )(_sc_body)


def kernel(query, value, ref_2d, spatial_shapes, level_start_index,
           W_off, b_off, W_attn, b_attn, W_val, b_val, W_out, b_out):
    q2 = query.reshape(_ROWS, _D)
    v2 = value.reshape(_ROWS, _D)
    r2 = ref_2d.reshape(_ROWS, 2)

    wo = W_off.reshape(_NH, _NP, 2, _D)
    wcat = jnp.concatenate(
        [wo[:, :, 0, :].reshape(32, _D),
         wo[:, :, 1, :].reshape(32, _D),
         W_attn], axis=0).T
    bo = b_off.reshape(_NH, _NP, 2)
    bcat = jnp.concatenate(
        [bo[:, :, 0].reshape(32), bo[:, :, 1].reshape(32), b_attn],
        axis=0).reshape(1, 96)
    smat = jnp.asarray(
        np.kron(np.eye(_NH, dtype=np.float32),
                np.ones((_NP, _NP), dtype=np.float32)))

    table = _mm_bias(v2, W_val.T, b_val.reshape(1, _D))
    idx, cw = _prep(q2, r2, wcat, bcat, smat)
    sampled = _sc_gather_fn()(table.reshape(_ROWS * _NH, _HD), idx, cw)
    out = _mm_bias_res(sampled.reshape(_ROWS, _D), W_out.T,
                       b_out.reshape(1, _D), q2)
    return out.reshape(_BS, _Q, _D)
